# baseline (speedup 1.0000x reference)
import functools

import numpy as np
import jax
import jax.numpy as jnp
from jax import lax
from jax.experimental import pallas as pl
from jax.experimental.pallas import tpu as pltpu
from jax.experimental.pallas import tpu_sc as plsc

_MAX_ITER = 100
_T_PAD = 128
_C = 16
_B, _N, _R = 8, 128, 48
_NW = 32
_BSC = 2
_JPW = _BSC * _N // _NW


def _universal_seqs() -> np.ndarray:
    table = np.zeros((8, _T_PAD), np.float32)
    for row, m1_0 in ((0, np.float32(1.0)), (3, np.float32(0.0))):
        m1 = m1_0
        m2 = np.float32(1.0) - m1_0
        for t in range(1, _MAX_ITER):
            table[row, t] = m1
            table[row + 1, t] = m2
            table[row + 2, t] = max(m1, m2)
            s_is_h1 = not (m1 >= m2)
            step = np.float32(2.0 / (t + 2))
            sig1 = np.float32(1.0) if s_is_h1 else np.float32(0.0)
            sig2 = np.float32(1.0) - sig1
            m1 = np.float32(m1 + step * (sig1 - m1))
            m2 = np.float32(m2 + step * (sig2 - m2))
    return table


_SEQ_TABLE = _universal_seqs()


def _sc_stats_kernel(sarc_hbm, srel_hbm, arcs_hbm, rels_hbm, out_hbm,
                     col0_v, col1_v, sa_v, arcs_v, rels_v, stats_v,
                     sem0, sem1, semp):
    wid = lax.axis_index("s") * 2 + lax.axis_index("c")
    b = wid // (_N // _JPW)
    j0 = (wid % (_N // _JPW)) * _JPW

    pltpu.async_copy(srel_hbm.at[b, j0], col0_v, sem0)
    pltpu.async_copy(srel_hbm.at[b, j0 + 1], col1_v, sem1)
    pltpu.async_copy(sarc_hbm.at[b, pl.ds(j0, _JPW)], sa_v, semp)
    pltpu.async_copy(arcs_hbm.at[b, pl.ds(j0, _JPW)], arcs_v, semp)
    pltpu.async_copy(rels_hbm.at[b, pl.ds(j0, _JPW)], rels_v, semp)
    pltpu.make_async_copy(sarc_hbm.at[b, pl.ds(j0, _JPW)], sa_v, semp).wait()
    pltpu.make_async_copy(arcs_hbm.at[b, pl.ds(j0, _JPW)], arcs_v, semp).wait()
    pltpu.make_async_copy(rels_hbm.at[b, pl.ds(j0, _JPW)], rels_v, semp).wait()

    lane = lax.iota(jnp.int32, 16)
    fzero = jnp.float32(0.0)
    neg = jnp.float32(-jnp.inf)
    bigf = jnp.float32(10_000.0)

    def _bfly(x, op):
        for sh in (8, 4, 2, 1):
            x = op(x, x.at[lane ^ sh].get(mode="promise_in_bounds"))
        return x

    allmax = lambda x: _bfly(x, jnp.maximum)
    allmin = lambda x: _bfly(x, jnp.minimum)
    allsum = lambda x: _bfly(x, lax.add)

    def compute_col(jl, col_v):
        j = j0 + jl

        jlvec0 = lane * 0 + jl
        gt = plsc.load_gather(arcs_v, [jlvec0])
        rl = plsc.load_gather(rels_v, [jlvec0])

        jlvec = jnp.full((16,), 0, jnp.int32) + jl
        idx_hs = [hg * 16 + lane for hg in range(8)]
        accs = [None] * 8
        for c in range(6):
            idxs = [jnp.where(lane + (c * 8 + k) >= _R,
                              lane + (c * 8 + k - _R), lane + (c * 8 + k))
                    for k in range(8)]
            for hg in range(8):
                gs = [plsc.load_gather(col_v, [idx_hs[hg], ii]) for ii in idxs]
                while len(gs) > 1:
                    gs = [jnp.maximum(gs[k], gs[k + 1])
                          for k in range(0, len(gs) - 1, 2)] \
                         + ([gs[-1]] if len(gs) % 2 else [])
                accs[hg] = gs[0] if accs[hg] is None \
                    else jnp.maximum(accs[hg], gs[0])
        sa_vecs = [plsc.load_gather(sa_v, [jlvec, idx_hs[hg]]) for hg in range(8)]
        a_vecs = [sa_vecs[hg] + accs[hg] for hg in range(8)]

        mvec = a_vecs[0]
        for hg in range(1, 8):
            mvec = jnp.maximum(mvec, a_vecs[hg])
        a1 = allmax(mvec)
        cand = jnp.full((16,), bigf, jnp.float32)
        for hg in range(8):
            idxf = (hg * 16 + lane).astype(jnp.float32)
            cand = jnp.minimum(cand, jnp.where(a_vecs[hg] == a1, idxf, bigf))
        h1 = allmin(cand).astype(jnp.int32)

        cand2 = jnp.full((16,), bigf, jnp.float32)
        masked = []
        for hg in range(8):
            idx_h = hg * 16 + lane
            masked.append(jnp.where(idx_h == h1, neg, a_vecs[hg]))
        m2vec = masked[0]
        for hg in range(1, 8):
            m2vec = jnp.maximum(m2vec, masked[hg])
        a2 = allmax(m2vec)
        for hg in range(8):
            idxf = (hg * 16 + lane).astype(jnp.float32)
            cand2 = jnp.minimum(cand2, jnp.where(masked[hg] == a2, idxf, bigf))
        h2 = allmin(cand2).astype(jnp.int32)

        d = j - 1
        accd = jnp.full((16,), fzero, jnp.float32)
        for hg in range(8):
            idx_h = hg * 16 + lane
            accd = accd + jnp.where(idx_h == d, a_vecs[hg], fzero)
        a_d = allsum(accd)
        sa_gt = plsc.load_gather(sa_v, [jlvec, gt])

        vks = []
        for k in range(3):
            xk = plsc.load_gather(col_v, [gt, k * 16 + lane])
            vks.append(sa_gt + xk)
        accG = jnp.full((16,), fzero, jnp.float32)
        pmax = jnp.full((16,), neg, jnp.float32)
        pens = []
        for k in range(3):
            ridx = k * 16 + lane
            accG = accG + jnp.where(ridx == rl, vks[k], fzero)
            pen = jnp.where(ridx == rl, vks[k], vks[k] + jnp.float32(1.0))
            pens.append(pen)
            pmax = jnp.maximum(pmax, pen)
        g = allsum(accG)
        penmax = allmax(pmax)
        candr = jnp.full((16,), bigf, jnp.float32)
        for k in range(3):
            ridxf = (k * 16 + lane).astype(jnp.float32)
            candr = jnp.minimum(candr, jnp.where(pens[k] == penmax, ridxf, bigf))
        rhat = allmin(candr).astype(jnp.int32)
        accv = jnp.full((16,), fzero, jnp.float32)
        for k in range(3):
            ridx = k * 16 + lane
            accv = accv + jnp.where(ridx == rhat, vks[k], fzero)
        vt = allsum(accv)

        one = jnp.float32(1.0)
        cm = jnp.where(j >= 1, one, fzero)
        case2 = (a1 - one) < a2
        startb = h1 == d
        c2f = jnp.where(case2, one, fzero)
        m1_ = (one - c2f) * cm
        m2a = c2f * jnp.where(startb, fzero, one) * cm
        m2b = c2f * jnp.where(startb, one, fzero) * cm
        vh1 = jnp.where(h1 == gt, vt, a1)
        vh2 = jnp.where(h2 == gt, vt, a2)
        vd = jnp.where(gt == d, vt, a_d)

        vals = [
            (one - a_d) * cm, (one - a1) * m1_,
            m2a, a1 * m2a, a2 * m2a,
            m2b, a1 * m2b, a2 * m2b,
            vh1 * m1_, vh1 * m2a, vh2 * m2a, vh1 * m2b, vh2 * m2b,
            vd * cm, g * cm,
        ]
        svec = jnp.full((16,), fzero, jnp.float32)
        for kk, val in enumerate(vals):
            svec = jnp.where(lane == kk, val, svec)
        stats_v[pl.ds(jl * _C, _C)] = svec

    def pair_body(i, carry):
        for ph, (buf, sem) in enumerate(((col0_v, sem0), (col1_v, sem1))):
            jl = i * 2 + ph
            pltpu.make_async_copy(srel_hbm.at[b, j0], buf, sem).wait()
            compute_col(jl, buf)
            nxt = jl + 2

            @pl.when(nxt < _JPW)
            def _refill():
                pltpu.async_copy(srel_hbm.at[b, j0 + nxt], buf, sem)
        return carry

    lax.fori_loop(0, _JPW // 2, pair_body, jnp.int32(0))
    pltpu.sync_copy(stats_v, out_hbm.at[pl.ds((b * _N + j0) * _C, _JPW * _C)])


def _tc_stats_kernel(sarc_ref, srel_ref, arcs_ref, rels_ref, out_ref):
    bg = pl.program_id(0) + _BSC
    n = sarc_ref.shape[-1]
    r = srel_ref.shape[-1]

    sa = sarc_ref[0]
    sr = srel_ref[0]

    jiota = jax.lax.broadcasted_iota(jnp.int32, (n,), 0)
    hiota2 = jax.lax.broadcasted_iota(jnp.int32, (n, n), 1)
    riota2 = jax.lax.broadcasted_iota(jnp.int32, (n, r), 1)

    rowmax = jnp.max(sr, axis=-1)
    acol = sa + rowmax

    a1 = jnp.max(acol, axis=1)
    h1 = jnp.argmax(acol, axis=1).astype(jnp.int32)
    neg = jnp.float32(-jnp.inf)
    am = jnp.where(hiota2 == h1[:, None], neg, acol)
    a2 = jnp.max(am, axis=1)
    h2 = jnp.argmax(am, axis=1).astype(jnp.int32)

    d = jiota - 1
    a_d = jnp.sum(jnp.where(hiota2 == d[:, None], acol, 0.0), axis=1)

    gt = arcs_ref[bg, :]
    rl = rels_ref[bg, :]

    ohg = hiota2 == gt[:, None]
    sa_gt = jnp.sum(jnp.where(ohg, sa, 0.0), axis=1)
    sr_gt = jax.lax.dot_general(
        ohg.astype(jnp.float32), sr,
        dimension_numbers=(((1,), (1,)), ((0,), (0,))),
        precision=jax.lax.Precision.HIGHEST)
    v = sa_gt[:, None] + sr_gt

    ohr = riota2 == rl[:, None]
    g = jnp.sum(jnp.where(ohr, v, 0.0), axis=1)
    pen = v + (1.0 - ohr.astype(jnp.float32))
    rhat = jnp.argmax(pen, axis=1).astype(jnp.int32)
    vt = jnp.sum(jnp.where(riota2 == rhat[:, None], v, 0.0), axis=1)

    vh1 = jnp.where(h1 == gt, vt, a1)
    vh2 = jnp.where(h2 == gt, vt, a2)
    vd = jnp.where(gt == d, vt, a_d)

    case2 = (a1 - 1.0) < a2
    startb = h1 == d
    colmask = jiota >= 1
    f = lambda m: m.astype(jnp.float32)
    m1_ = f((~case2) & colmask)
    m2a = f(case2 & (~startb) & colmask)
    m2b = f(case2 & startb & colmask)
    cm = f(colmask)

    out_ref[0] = jnp.stack([
        (1.0 - a_d) * cm, (1.0 - a1) * m1_,
        m2a, a1 * m2a, a2 * m2a,
        m2b, a1 * m2b, a2 * m2b,
        vh1 * m1_, vh1 * m2a, vh2 * m2a, vh1 * m2b, vh2 * m2b,
        vd * cm, g * cm, cm * 0.0,
    ])


def _finalize_kernel(seq_ref, scs_ref, tcs_ref, out_ref):
    s = (jnp.sum(scs_ref[...], axis=(0, 1))
         + jnp.sum(tcs_ref[...], axis=(0, 2)))
    seq = seq_ref[...]
    tio = jax.lax.broadcasted_iota(jnp.int32, (_T_PAD,), 0)
    obj = (s[1] + s[2] * seq[2] - s[3] * seq[0] - s[4] * seq[1]
           + s[5] * seq[5] - s[6] * seq[3] - s[7] * seq[4])
    obj = jnp.where(tio == 0, s[0], obj)
    obj = jnp.where(tio >= _MAX_ITER, jnp.float32(jnp.inf), obj)
    tstar = jnp.argmin(obj).astype(jnp.int32)
    sel = lambda row: jnp.sum(jnp.where(tio == tstar, row, 0.0))
    m1a, m2a = sel(seq[0]), sel(seq[1])
    m1b, m2b = sel(seq[3]), sel(seq[4])
    adv = s[8] + m1a * s[9] + m2a * s[10] + m1b * s[11] + m2b * s[12]
    adv = jnp.where(tstar == 0, s[13], adv)
    out_ref[...] = jnp.reshape((adv - s[14]) / 8.0, (1, 1))


@jax.jit
def _adv_loss(s_arc, s_rel, arcs, rels):
    n, r = _N, _R
    sc_stats = functools.partial(
        pl.kernel,
        mesh=plsc.VectorSubcoreMesh(core_axis_name="c", subcore_axis_name="s"),
        out_type=jax.ShapeDtypeStruct((_BSC * _N * _C,), jnp.float32),
        compiler_params=pltpu.CompilerParams(needs_layout_passes=False),
        scratch_types=[
            pltpu.VMEM((_N, _R), jnp.float32),
            pltpu.VMEM((_N, _R), jnp.float32),
            pltpu.VMEM((_JPW, _N), jnp.float32),
            pltpu.VMEM((_JPW,), jnp.int32),
            pltpu.VMEM((_JPW,), jnp.int32),
            pltpu.VMEM((_JPW * _C,), jnp.float32),
            pltpu.SemaphoreType.DMA,
            pltpu.SemaphoreType.DMA,
            pltpu.SemaphoreType.DMA,
        ],
    )(_sc_stats_kernel)
    scs = sc_stats(s_arc, s_rel, arcs, rels).reshape(_BSC, _N, _C)

    tcs = pl.pallas_call(
        _tc_stats_kernel,
        grid=(_B - _BSC,),
        in_specs=[
            pl.BlockSpec((1, n, n), lambda i: (i + _BSC, 0, 0)),
            pl.BlockSpec((1, n, n, r), lambda i: (i + _BSC, 0, 0, 0)),
            pl.BlockSpec((_B, n), lambda i: (0, 0)),
            pl.BlockSpec((_B, n), lambda i: (0, 0)),
        ],
        out_specs=pl.BlockSpec((1, _C, n), lambda i: (i, 0, 0)),
        out_shape=jax.ShapeDtypeStruct((_B - _BSC, _C, n), jnp.float32),
    )(s_arc, s_rel, arcs, rels)

    seq = jnp.asarray(_SEQ_TABLE)
    out = pl.pallas_call(
        _finalize_kernel,
        in_specs=[
            pl.BlockSpec((8, _T_PAD), lambda: (0, 0)),
            pl.BlockSpec((_BSC, _N, _C), lambda: (0, 0, 0)),
            pl.BlockSpec((_B - _BSC, _C, _N), lambda: (0, 0, 0)),
        ],
        out_specs=pl.BlockSpec((1, 1), lambda: (0, 0)),
        out_shape=jax.ShapeDtypeStruct((1, 1), jnp.float32),
    )(seq, scs, tcs)
    return jnp.reshape(out, ())


def kernel(s_arc, arcs, s_rel, rels, mask, lambd):
    del mask, lambd
    return _adv_loss(s_arc.astype(jnp.float32), s_rel.astype(jnp.float32),
                     arcs.astype(jnp.int32), rels.astype(jnp.int32))


# speedup vs baseline: 1.0582x; 1.0582x over previous
import functools

import numpy as np
import jax
import jax.numpy as jnp
from jax import lax
from jax.experimental import pallas as pl
from jax.experimental.pallas import tpu as pltpu
from jax.experimental.pallas import tpu_sc as plsc

_MAX_ITER = 100
_T_PAD = 128
_C = 16
_B, _N, _R = 8, 128, 48
_NW = 32
_BSC = 4
_JPW = _BSC * _N // _NW


def _universal_seqs() -> np.ndarray:
    table = np.zeros((8, _T_PAD), np.float32)
    for row, m1_0 in ((0, np.float32(1.0)), (3, np.float32(0.0))):
        m1 = m1_0
        m2 = np.float32(1.0) - m1_0
        for t in range(1, _MAX_ITER):
            table[row, t] = m1
            table[row + 1, t] = m2
            table[row + 2, t] = max(m1, m2)
            s_is_h1 = not (m1 >= m2)
            step = np.float32(2.0 / (t + 2))
            sig1 = np.float32(1.0) if s_is_h1 else np.float32(0.0)
            sig2 = np.float32(1.0) - sig1
            m1 = np.float32(m1 + step * (sig1 - m1))
            m2 = np.float32(m2 + step * (sig2 - m2))
    return table


_SEQ_TABLE = _universal_seqs()


def _sc_stats_kernel(sarc_hbm, srel_hbm, arcs_hbm, rels_hbm, out_hbm,
                     col0_v, col1_v, sa_v, arcs_v, rels_v, stats_v,
                     sem0, sem1, semp):
    wid = lax.axis_index("s") * 2 + lax.axis_index("c")
    b = wid // (_N // _JPW)
    j0 = (wid % (_N // _JPW)) * _JPW

    pltpu.async_copy(srel_hbm.at[b, j0], col0_v, sem0)
    pltpu.async_copy(srel_hbm.at[b, j0 + 1], col1_v, sem1)
    pltpu.async_copy(sarc_hbm.at[b, pl.ds(j0, _JPW)], sa_v, semp)
    pltpu.async_copy(arcs_hbm.at[b, pl.ds(j0, _JPW)], arcs_v, semp)
    pltpu.async_copy(rels_hbm.at[b, pl.ds(j0, _JPW)], rels_v, semp)
    pltpu.make_async_copy(sarc_hbm.at[b, pl.ds(j0, _JPW)], sa_v, semp).wait()
    pltpu.make_async_copy(arcs_hbm.at[b, pl.ds(j0, _JPW)], arcs_v, semp).wait()
    pltpu.make_async_copy(rels_hbm.at[b, pl.ds(j0, _JPW)], rels_v, semp).wait()

    lane = lax.iota(jnp.int32, 16)
    fzero = jnp.float32(0.0)
    neg = jnp.float32(-jnp.inf)
    bigf = jnp.float32(10_000.0)

    def _bfly(x, op):
        for sh in (8, 4, 2, 1):
            x = op(x, x.at[lane ^ sh].get(mode="promise_in_bounds"))
        return x

    allmax = lambda x: _bfly(x, jnp.maximum)
    allmin = lambda x: _bfly(x, jnp.minimum)
    allsum = lambda x: _bfly(x, lax.add)

    def compute_col(jl, col_v):
        j = j0 + jl

        jlvec0 = lane * 0 + jl
        gt = plsc.load_gather(arcs_v, [jlvec0])
        rl = plsc.load_gather(rels_v, [jlvec0])

        jlvec = jnp.full((16,), 0, jnp.int32) + jl
        idx_hs = [hg * 16 + lane for hg in range(8)]
        accs = [None] * 8
        for c in range(6):
            idxs = [jnp.where(lane + (c * 8 + k) >= _R,
                              lane + (c * 8 + k - _R), lane + (c * 8 + k))
                    for k in range(8)]
            for hg in range(8):
                gs = [plsc.load_gather(col_v, [idx_hs[hg], ii]) for ii in idxs]
                while len(gs) > 1:
                    gs = [jnp.maximum(gs[k], gs[k + 1])
                          for k in range(0, len(gs) - 1, 2)] \
                         + ([gs[-1]] if len(gs) % 2 else [])
                accs[hg] = gs[0] if accs[hg] is None \
                    else jnp.maximum(accs[hg], gs[0])
        sa_vecs = [plsc.load_gather(sa_v, [jlvec, idx_hs[hg]]) for hg in range(8)]
        a_vecs = [sa_vecs[hg] + accs[hg] for hg in range(8)]

        mvec = a_vecs[0]
        for hg in range(1, 8):
            mvec = jnp.maximum(mvec, a_vecs[hg])
        a1 = allmax(mvec)
        cand = jnp.full((16,), bigf, jnp.float32)
        for hg in range(8):
            idxf = (hg * 16 + lane).astype(jnp.float32)
            cand = jnp.minimum(cand, jnp.where(a_vecs[hg] == a1, idxf, bigf))
        h1 = allmin(cand).astype(jnp.int32)

        cand2 = jnp.full((16,), bigf, jnp.float32)
        masked = []
        for hg in range(8):
            idx_h = hg * 16 + lane
            masked.append(jnp.where(idx_h == h1, neg, a_vecs[hg]))
        m2vec = masked[0]
        for hg in range(1, 8):
            m2vec = jnp.maximum(m2vec, masked[hg])
        a2 = allmax(m2vec)
        for hg in range(8):
            idxf = (hg * 16 + lane).astype(jnp.float32)
            cand2 = jnp.minimum(cand2, jnp.where(masked[hg] == a2, idxf, bigf))
        h2 = allmin(cand2).astype(jnp.int32)

        d = j - 1
        accd = jnp.full((16,), fzero, jnp.float32)
        for hg in range(8):
            idx_h = hg * 16 + lane
            accd = accd + jnp.where(idx_h == d, a_vecs[hg], fzero)
        a_d = allsum(accd)
        sa_gt = plsc.load_gather(sa_v, [jlvec, gt])

        vks = []
        for k in range(3):
            xk = plsc.load_gather(col_v, [gt, k * 16 + lane])
            vks.append(sa_gt + xk)
        accG = jnp.full((16,), fzero, jnp.float32)
        pmax = jnp.full((16,), neg, jnp.float32)
        pens = []
        for k in range(3):
            ridx = k * 16 + lane
            accG = accG + jnp.where(ridx == rl, vks[k], fzero)
            pen = jnp.where(ridx == rl, vks[k], vks[k] + jnp.float32(1.0))
            pens.append(pen)
            pmax = jnp.maximum(pmax, pen)
        g = allsum(accG)
        penmax = allmax(pmax)
        candr = jnp.full((16,), bigf, jnp.float32)
        for k in range(3):
            ridxf = (k * 16 + lane).astype(jnp.float32)
            candr = jnp.minimum(candr, jnp.where(pens[k] == penmax, ridxf, bigf))
        rhat = allmin(candr).astype(jnp.int32)
        accv = jnp.full((16,), fzero, jnp.float32)
        for k in range(3):
            ridx = k * 16 + lane
            accv = accv + jnp.where(ridx == rhat, vks[k], fzero)
        vt = allsum(accv)

        one = jnp.float32(1.0)
        cm = jnp.where(j >= 1, one, fzero)
        case2 = (a1 - one) < a2
        startb = h1 == d
        c2f = jnp.where(case2, one, fzero)
        m1_ = (one - c2f) * cm
        m2a = c2f * jnp.where(startb, fzero, one) * cm
        m2b = c2f * jnp.where(startb, one, fzero) * cm
        vh1 = jnp.where(h1 == gt, vt, a1)
        vh2 = jnp.where(h2 == gt, vt, a2)
        vd = jnp.where(gt == d, vt, a_d)

        vals = [
            (one - a_d) * cm, (one - a1) * m1_,
            m2a, a1 * m2a, a2 * m2a,
            m2b, a1 * m2b, a2 * m2b,
            vh1 * m1_, vh1 * m2a, vh2 * m2a, vh1 * m2b, vh2 * m2b,
            vd * cm, g * cm,
        ]
        svec = jnp.full((16,), fzero, jnp.float32)
        for kk, val in enumerate(vals):
            svec = jnp.where(lane == kk, val, svec)
        stats_v[pl.ds(jl * _C, _C)] = svec

    def pair_body(i, carry):
        for ph, (buf, sem) in enumerate(((col0_v, sem0), (col1_v, sem1))):
            jl = i * 2 + ph
            pltpu.make_async_copy(srel_hbm.at[b, j0], buf, sem).wait()
            compute_col(jl, buf)
            nxt = jl + 2

            @pl.when(nxt < _JPW)
            def _refill():
                pltpu.async_copy(srel_hbm.at[b, j0 + nxt], buf, sem)
        return carry

    lax.fori_loop(0, _JPW // 2, pair_body, jnp.int32(0))
    pltpu.sync_copy(stats_v, out_hbm.at[pl.ds((b * _N + j0) * _C, _JPW * _C)])


def _tc_stats_kernel(sarc_ref, srel_ref, arcs_ref, rels_ref, out_ref):
    bg = pl.program_id(0) + _BSC
    n = sarc_ref.shape[-1]
    r = srel_ref.shape[-1]

    sa = sarc_ref[0]
    sr = srel_ref[0]

    jiota = jax.lax.broadcasted_iota(jnp.int32, (n,), 0)
    hiota2 = jax.lax.broadcasted_iota(jnp.int32, (n, n), 1)
    riota2 = jax.lax.broadcasted_iota(jnp.int32, (n, r), 1)

    rowmax = jnp.max(sr, axis=-1)
    acol = sa + rowmax

    a1 = jnp.max(acol, axis=1)
    h1 = jnp.argmax(acol, axis=1).astype(jnp.int32)
    neg = jnp.float32(-jnp.inf)
    am = jnp.where(hiota2 == h1[:, None], neg, acol)
    a2 = jnp.max(am, axis=1)
    h2 = jnp.argmax(am, axis=1).astype(jnp.int32)

    d = jiota - 1
    a_d = jnp.sum(jnp.where(hiota2 == d[:, None], acol, 0.0), axis=1)

    gt = arcs_ref[bg, :]
    rl = rels_ref[bg, :]

    ohg = hiota2 == gt[:, None]
    sa_gt = jnp.sum(jnp.where(ohg, sa, 0.0), axis=1)
    sr_gt = jax.lax.dot_general(
        ohg.astype(jnp.float32), sr,
        dimension_numbers=(((1,), (1,)), ((0,), (0,))),
        precision=jax.lax.Precision.HIGHEST)
    v = sa_gt[:, None] + sr_gt

    ohr = riota2 == rl[:, None]
    g = jnp.sum(jnp.where(ohr, v, 0.0), axis=1)
    pen = v + (1.0 - ohr.astype(jnp.float32))
    rhat = jnp.argmax(pen, axis=1).astype(jnp.int32)
    vt = jnp.sum(jnp.where(riota2 == rhat[:, None], v, 0.0), axis=1)

    vh1 = jnp.where(h1 == gt, vt, a1)
    vh2 = jnp.where(h2 == gt, vt, a2)
    vd = jnp.where(gt == d, vt, a_d)

    case2 = (a1 - 1.0) < a2
    startb = h1 == d
    colmask = jiota >= 1
    f = lambda m: m.astype(jnp.float32)
    m1_ = f((~case2) & colmask)
    m2a = f(case2 & (~startb) & colmask)
    m2b = f(case2 & startb & colmask)
    cm = f(colmask)

    out_ref[0] = jnp.stack([
        (1.0 - a_d) * cm, (1.0 - a1) * m1_,
        m2a, a1 * m2a, a2 * m2a,
        m2b, a1 * m2b, a2 * m2b,
        vh1 * m1_, vh1 * m2a, vh2 * m2a, vh1 * m2b, vh2 * m2b,
        vd * cm, g * cm, cm * 0.0,
    ])


def _finalize_kernel(seq_ref, scs_ref, tcs_ref, out_ref):
    s = (jnp.sum(scs_ref[...], axis=(0, 1))
         + jnp.sum(tcs_ref[...], axis=(0, 2)))
    seq = seq_ref[...]
    tio = jax.lax.broadcasted_iota(jnp.int32, (_T_PAD,), 0)
    obj = (s[1] + s[2] * seq[2] - s[3] * seq[0] - s[4] * seq[1]
           + s[5] * seq[5] - s[6] * seq[3] - s[7] * seq[4])
    obj = jnp.where(tio == 0, s[0], obj)
    obj = jnp.where(tio >= _MAX_ITER, jnp.float32(jnp.inf), obj)
    tstar = jnp.argmin(obj).astype(jnp.int32)
    sel = lambda row: jnp.sum(jnp.where(tio == tstar, row, 0.0))
    m1a, m2a = sel(seq[0]), sel(seq[1])
    m1b, m2b = sel(seq[3]), sel(seq[4])
    adv = s[8] + m1a * s[9] + m2a * s[10] + m1b * s[11] + m2b * s[12]
    adv = jnp.where(tstar == 0, s[13], adv)
    out_ref[...] = jnp.reshape((adv - s[14]) / 8.0, (1, 1))


@jax.jit
def _adv_loss(s_arc, s_rel, arcs, rels):
    n, r = _N, _R
    sc_stats = functools.partial(
        pl.kernel,
        mesh=plsc.VectorSubcoreMesh(core_axis_name="c", subcore_axis_name="s"),
        out_type=jax.ShapeDtypeStruct((_BSC * _N * _C,), jnp.float32),
        compiler_params=pltpu.CompilerParams(needs_layout_passes=False),
        scratch_types=[
            pltpu.VMEM((_N, _R), jnp.float32),
            pltpu.VMEM((_N, _R), jnp.float32),
            pltpu.VMEM((_JPW, _N), jnp.float32),
            pltpu.VMEM((_JPW,), jnp.int32),
            pltpu.VMEM((_JPW,), jnp.int32),
            pltpu.VMEM((_JPW * _C,), jnp.float32),
            pltpu.SemaphoreType.DMA,
            pltpu.SemaphoreType.DMA,
            pltpu.SemaphoreType.DMA,
        ],
    )(_sc_stats_kernel)
    scs = sc_stats(s_arc, s_rel, arcs, rels).reshape(_BSC, _N, _C)

    tcs = pl.pallas_call(
        _tc_stats_kernel,
        grid=(_B - _BSC,),
        in_specs=[
            pl.BlockSpec((1, n, n), lambda i: (i + _BSC, 0, 0)),
            pl.BlockSpec((1, n, n, r), lambda i: (i + _BSC, 0, 0, 0)),
            pl.BlockSpec((_B, n), lambda i: (0, 0)),
            pl.BlockSpec((_B, n), lambda i: (0, 0)),
        ],
        out_specs=pl.BlockSpec((1, _C, n), lambda i: (i, 0, 0)),
        out_shape=jax.ShapeDtypeStruct((_B - _BSC, _C, n), jnp.float32),
    )(s_arc, s_rel, arcs, rels)

    seq = jnp.asarray(_SEQ_TABLE)
    out = pl.pallas_call(
        _finalize_kernel,
        in_specs=[
            pl.BlockSpec((8, _T_PAD), lambda: (0, 0)),
            pl.BlockSpec((_BSC, _N, _C), lambda: (0, 0, 0)),
            pl.BlockSpec((_B - _BSC, _C, _N), lambda: (0, 0, 0)),
        ],
        out_specs=pl.BlockSpec((1, 1), lambda: (0, 0)),
        out_shape=jax.ShapeDtypeStruct((1, 1), jnp.float32),
    )(seq, scs, tcs)
    return jnp.reshape(out, ())


def kernel(s_arc, arcs, s_rel, rels, mask, lambd):
    del mask, lambd
    return _adv_loss(s_arc.astype(jnp.float32), s_rel.astype(jnp.float32),
                     arcs.astype(jnp.int32), rels.astype(jnp.int32))
